# Initial kernel scaffold; baseline (speedup 1.0000x reference)
import jax
import jax.numpy as jnp
from jax.experimental import pallas as pl


def kernel(x, edge_index, edge_attr, W_msg, b_msg, W_edge, b_edge, W_upd, b_upd):
    raise NotImplementedError("write your pallas kernel here")


# speedup vs baseline: 3.6717x; 3.6717x over previous
import functools

import jax
import jax.numpy as jnp
from jax import lax
from jax.experimental import pallas as pl
from jax.experimental.pallas import tpu as pltpu
from jax.experimental.pallas import tpu_sc as plsc

N = 10000
E = 320000
D_IN = 128
D_OUT = 128
D_EDGE = 16

NC = 2
NS = 16
NW = NC * NS
C = 128
NCHUNKS = E // C
RPT = 624
TAIL = N - NS * RPT
Z = 48

_mesh = plsc.VectorSubcoreMesh(core_axis_name="c", subcore_axis_name="s")


@functools.partial(
    pl.kernel,
    out_type=(
        jax.ShapeDtypeStruct((NC, N, D_IN), jnp.float32),
        jax.ShapeDtypeStruct((NC, N, D_EDGE), jnp.float32),
        jax.ShapeDtypeStruct((NC, N, D_EDGE), jnp.float32),
    ),
    mesh=_mesh,
    scratch_types=[
        pltpu.VMEM_SHARED((N, D_IN), jnp.float32),
        pltpu.VMEM((C, D_IN), jnp.float32),
        pltpu.VMEM((C, D_EDGE), jnp.float32),
        pltpu.VMEM((C,), jnp.int32),
        pltpu.VMEM((C,), jnp.int32),
        pltpu.VMEM((Z,), jnp.int32),
        pltpu.VMEM((TAIL,), jnp.int32),
        pltpu.SemaphoreType.DMA,
    ],
)
def _sc_segment_sums(x_hbm, row_hbm, col_hbm, ea_hbm, zero_hbm,
                     px_hbm, pe_hbm, pc_hbm,
                     acc,
                     rows_v, attr_v, rowi_v, coli_v, zidx_v, tidx_v, sem):
    c = lax.axis_index("c")
    s = lax.axis_index("s")
    w = c * NS + s
    base_r = s * RPT

    def _set_zidx(base, clamp):
        for k in range(Z // 16):
            v = base + k * 16 + lax.iota(jnp.int32, 16)
            if clamp:
                v = jnp.minimum(v, N - 1)
            zidx_v[pl.ds(16 * k, 16)] = v

    def _zero_acc():
        for j in range(14):
            _set_zidx(base_r + j * Z, clamp=True)
            pltpu.async_copy(rows_v.at[pl.ds(0, Z)], acc.at[zidx_v],
                             sem).wait()

    nmine = (NCHUNKS - w + NW - 1) // NW

    pltpu.sync_copy(zero_hbm, rows_v)
    _zero_acc()
    plsc.subcore_barrier()

    @pl.loop(0, nmine)
    def _chunk_a(i):
        off = (w + i * NW) * C
        pltpu.sync_copy(row_hbm.at[pl.ds(off, C)], rowi_v)
        pltpu.sync_copy(col_hbm.at[pl.ds(off, C)], coli_v)
        pltpu.async_copy(x_hbm.at[coli_v], rows_v, sem).wait()
        pltpu.async_copy(rows_v, acc.at[rowi_v], sem, add=True).wait()

    plsc.subcore_barrier()

    for j in range(13):
        _set_zidx(base_r + j * Z, clamp=False)
        r0 = base_r + j * Z
        pltpu.async_copy(acc.at[zidx_v], rows_v.at[pl.ds(0, Z)], sem).wait()
        pltpu.sync_copy(rows_v.at[pl.ds(0, Z)], px_hbm.at[c, pl.ds(r0, Z)])

    @pl.when(s == NS - 1)
    def _export_tail_a():
        tidx_v[...] = NS * RPT + lax.iota(jnp.int32, TAIL)
        pltpu.async_copy(acc.at[tidx_v], rows_v.at[pl.ds(0, TAIL)],
                         sem).wait()
        pltpu.sync_copy(rows_v.at[pl.ds(0, TAIL)],
                        px_hbm.at[c, pl.ds(NS * RPT, TAIL)])

    plsc.subcore_barrier()

    pltpu.sync_copy(zero_hbm, rows_v)
    _zero_acc()

    onehot16 = jnp.where(lax.iota(jnp.int32, 16) == 0,
                         jnp.float32(1), jnp.float32(0))

    @pl.loop(0, C)
    def _ones_col(i):
        rows_v[i, pl.ds(16, 16)] = onehot16

    plsc.subcore_barrier()

    @pl.loop(0, nmine)
    def _chunk_b(i):
        off = (w + i * NW) * C
        pltpu.sync_copy(row_hbm.at[pl.ds(off, C)], rowi_v)
        pltpu.sync_copy(ea_hbm.at[pl.ds(off, C)], attr_v)

        @pl.loop(0, C)
        def _stage(k):
            rows_v[k, pl.ds(0, 16)] = attr_v[k, :]

        pltpu.async_copy(rows_v, acc.at[rowi_v], sem, add=True).wait()

    plsc.subcore_barrier()

    def _export_ec(r0, n):
        pltpu.async_copy(acc.at[zidx_v if n == Z else tidx_v],
                         rows_v.at[pl.ds(0, n)], sem).wait()

        @pl.loop(0, n)
        def _stage_e(k):
            attr_v[k, :] = rows_v[k, pl.ds(0, 16)]

        pltpu.sync_copy(attr_v.at[pl.ds(0, n)], pe_hbm.at[c, pl.ds(r0, n)])

        @pl.loop(0, n)
        def _stage_c(k):
            attr_v[k, :] = rows_v[k, pl.ds(16, 16)]

        pltpu.sync_copy(attr_v.at[pl.ds(0, n)], pc_hbm.at[c, pl.ds(r0, n)])

    for j in range(13):
        _set_zidx(base_r + j * Z, clamp=False)
        _export_ec(base_r + j * Z, Z)

    @pl.when(s == NS - 1)
    def _export_tail_b():
        tidx_v[...] = NS * RPT + lax.iota(jnp.int32, TAIL)
        _export_ec(NS * RPT, TAIL)


_B = 2000


def _tc_body(px_ref, pe_ref, pc_ref, x4_ref, wm_ref, we_ref, wu_ref,
             bme_ref, bu_ref, out_ref, scale_smem):
    i = pl.program_id(0)
    sumx = px_ref[0] + px_ref[1]
    sume = pe_ref[0] + pe_ref[1]
    cnt = pc_ref[0, :, 0:1] + pc_ref[1, :, 0:1]
    acc = jnp.dot(sumx, wm_ref[...], preferred_element_type=jnp.float32)
    acc = acc + jnp.dot(sume, we_ref[...], preferred_element_type=jnp.float32)
    acc = acc + cnt * bme_ref[...]
    mean = acc / (cnt + 1e-8)
    upd = jnp.dot(mean, wu_ref[...], preferred_element_type=jnp.float32)
    upd = upd + bu_ref[...]
    norm = jnp.sqrt(jnp.sum(upd * upd, axis=1, keepdims=True) + 1e-8)
    proj = upd * jnp.minimum(1.0, (1.0 - 1e-5) / norm)

    def _cr(r):
        def dist(u, v):
            return jnp.sqrt(jnp.sum((u - v) ** 2) + 1e-8)
        a, b, cc, d = r[0:1, :], r[1:2, :], r[2:3, :], r[3:4, :]
        return (dist(a, cc) * dist(b, d)) / (dist(a, d) * dist(b, cc) + 1e-8)

    @pl.when(i == 0)
    def _():
        cr_init = _cr(x4_ref[...])
        cr_cur = _cr(proj[0:4, :])
        scale_smem[0] = cr_init / (cr_cur + 1e-8)

    out_ref[...] = proj * scale_smem[0]


def _tc_epilogue(px, pe, pc, x4, W_msg, W_edge, W_upd, bme, bu):
    return pl.pallas_call(
        _tc_body,
        out_shape=jax.ShapeDtypeStruct((N, D_OUT), jnp.float32),
        grid=(N // _B,),
        in_specs=[
            pl.BlockSpec((NC, _B, D_IN), lambda i: (0, i, 0)),
            pl.BlockSpec((NC, _B, D_EDGE), lambda i: (0, i, 0)),
            pl.BlockSpec((NC, _B, D_EDGE), lambda i: (0, i, 0)),
            pl.BlockSpec((4, D_IN), lambda i: (0, 0)),
            pl.BlockSpec((D_IN, D_OUT), lambda i: (0, 0)),
            pl.BlockSpec((D_EDGE, D_OUT), lambda i: (0, 0)),
            pl.BlockSpec((D_OUT, D_OUT), lambda i: (0, 0)),
            pl.BlockSpec((1, D_OUT), lambda i: (0, 0)),
            pl.BlockSpec((1, D_OUT), lambda i: (0, 0)),
        ],
        out_specs=pl.BlockSpec((_B, D_OUT), lambda i: (i, 0)),
        scratch_shapes=[pltpu.SMEM((1,), jnp.float32)],
    )(px, pe, pc, x4, W_msg, W_edge, W_upd, bme, bu)


def kernel(x, edge_index, edge_attr, W_msg, b_msg, W_edge, b_edge, W_upd, b_upd):
    row = edge_index[0]
    col = edge_index[1]
    zero = jnp.zeros((C, D_IN), jnp.float32)
    px, pe, pc = _sc_segment_sums(x, row, col, edge_attr, zero)
    x4 = x[0:4]
    bme = (b_msg + b_edge).reshape(1, D_OUT)
    bu = b_upd.reshape(1, D_OUT)
    return _tc_epilogue(px, pe, pc, x4, W_msg, W_edge, W_upd, bme, bu)


# speedup vs baseline: 5.5597x; 1.5142x over previous
import functools

import jax
import jax.numpy as jnp
from jax import lax
from jax.experimental import pallas as pl
from jax.experimental.pallas import tpu as pltpu
from jax.experimental.pallas import tpu_sc as plsc

N = 10000
E = 320000
D_IN = 128
D_OUT = 128
D_EDGE = 16

NC = 2
NS = 16
NW = NC * NS
C = 80
NCHUNKS = E // C
CPW = NCHUNKS // NW
RPT = 624
TAIL = N - NS * RPT
Z = 48

_mesh = plsc.VectorSubcoreMesh(core_axis_name="c", subcore_axis_name="s")


@functools.partial(
    pl.kernel,
    out_type=(
        jax.ShapeDtypeStruct((NC, N, D_IN), jnp.float32),
        jax.ShapeDtypeStruct((NC, N, D_EDGE), jnp.float32),
        jax.ShapeDtypeStruct((NC, N, D_EDGE), jnp.float32),
    ),
    mesh=_mesh,
    scratch_types=[
        pltpu.VMEM_SHARED((N, D_IN), jnp.float32),
        pltpu.VMEM((2, C, D_IN), jnp.float32),
        pltpu.VMEM((2, C, D_EDGE), jnp.float32),
        pltpu.VMEM((4, C), jnp.int32),
        pltpu.VMEM((4, C), jnp.int32),
        pltpu.VMEM((Z,), jnp.int32),
        pltpu.VMEM((TAIL,), jnp.int32),
        [pltpu.SemaphoreType.DMA] * 4,
        [pltpu.SemaphoreType.DMA] * 2,
        [pltpu.SemaphoreType.DMA] * 2,
        pltpu.SemaphoreType.DMA,
    ],
)
def _sc_segment_sums(x_hbm, row_hbm, col_hbm, ea_hbm, zero_hbm,
                     px_hbm, pe_hbm, pc_hbm,
                     acc, rows_v, attr_v, rowi_v, coli_v, zidx_v, tidx_v,
                     semI, semG, semS, sem):
    cid = lax.axis_index("c")
    s = lax.axis_index("s")
    w = cid * NS + s
    base_r = s * RPT

    def _set_zidx(base, clamp):
        for k in range(Z // 16):
            v = base + k * 16 + lax.iota(jnp.int32, 16)
            if clamp:
                v = jnp.minimum(v, N - 1)
            zidx_v[pl.ds(16 * k, 16)] = v

    def _zero_acc():
        for j in range(14):
            _set_zidx(base_r + j * Z, clamp=True)
            pltpu.async_copy(rows_v.at[0, pl.ds(0, Z)], acc.at[zidx_v],
                             sem).wait()

    def _off(j):
        return (w + j * NW) * C

    pltpu.sync_copy(zero_hbm, rows_v.at[0])
    _zero_acc()
    plsc.subcore_barrier()

    def a_issue_idx(ib, j):
        pltpu.async_copy(row_hbm.at[pl.ds(_off(j), C)], rowi_v.at[ib],
                         semI[ib])
        pltpu.async_copy(col_hbm.at[pl.ds(_off(j), C)], coli_v.at[ib],
                         semI[ib])

    def a_wait_idx(ib, j):
        pltpu.make_async_copy(row_hbm.at[pl.ds(_off(j), C)], rowi_v.at[ib],
                              semI[ib]).wait()
        pltpu.make_async_copy(col_hbm.at[pl.ds(_off(j), C)], coli_v.at[ib],
                              semI[ib]).wait()

    def a_wait_scatter(rb, ib):
        pltpu.make_async_copy(rows_v.at[rb], acc.at[rowi_v.at[ib]],
                              semS[rb]).wait()

    a_issue_idx(0, 0)
    a_issue_idx(1, 1)

    @pl.loop(0, (CPW - 1) // 4)
    def _ring_a(g):
        for b in range(4):
            j = 4 * g + b
            rb, ib, ipb = b % 2, b, (b + 2) % 4
            a_wait_idx(ib, j)
            if b < 2:
                @pl.when(g >= 1)
                def _():
                    a_wait_scatter(rb, ipb)
            else:
                a_wait_scatter(rb, ipb)

            @pl.when(j + 2 <= CPW - 1)
            def _():
                a_issue_idx(ipb, j + 2)

            pltpu.async_copy(x_hbm.at[coli_v.at[ib]], rows_v.at[rb],
                             semG[rb]).wait()
            pltpu.async_copy(rows_v.at[rb], acc.at[rowi_v.at[ib]],
                             semS[rb], add=True)

    a_wait_idx(0, CPW - 1)
    a_wait_scatter(0, 2)
    pltpu.async_copy(x_hbm.at[coli_v.at[0]], rows_v.at[0], semG[0]).wait()
    pltpu.async_copy(rows_v.at[0], acc.at[rowi_v.at[0]], semS[0], add=True)
    a_wait_scatter(0, 0)
    a_wait_scatter(1, 3)

    plsc.subcore_barrier()

    for j in range(13):
        _set_zidx(base_r + j * Z, clamp=False)
        r0 = base_r + j * Z
        pltpu.async_copy(acc.at[zidx_v], rows_v.at[0, pl.ds(0, Z)],
                         sem).wait()
        pltpu.sync_copy(rows_v.at[0, pl.ds(0, Z)], px_hbm.at[cid, pl.ds(r0, Z)])

    @pl.when(s == NS - 1)
    def _export_tail_a():
        tidx_v[...] = NS * RPT + lax.iota(jnp.int32, TAIL)
        pltpu.async_copy(acc.at[tidx_v], rows_v.at[0, pl.ds(0, TAIL)],
                         sem).wait()
        pltpu.sync_copy(rows_v.at[0, pl.ds(0, TAIL)],
                        px_hbm.at[cid, pl.ds(NS * RPT, TAIL)])

    plsc.subcore_barrier()

    pltpu.sync_copy(zero_hbm, rows_v.at[0])
    _zero_acc()
    pltpu.sync_copy(zero_hbm, rows_v.at[1])

    onehot16 = jnp.where(lax.iota(jnp.int32, 16) == 0,
                         jnp.float32(1), jnp.float32(0))

    @pl.loop(0, C)
    def _ones_col(i):
        rows_v[0, i, pl.ds(16, 16)] = onehot16
        rows_v[1, i, pl.ds(16, 16)] = onehot16

    plsc.subcore_barrier()

    def b_issue_idx(ib, j):
        pltpu.async_copy(row_hbm.at[pl.ds(_off(j), C)], rowi_v.at[ib],
                         semI[ib])
        pltpu.async_copy(ea_hbm.at[pl.ds(_off(j), C)], attr_v.at[ib % 2],
                         semI[ib])

    def b_wait_idx(ib, j):
        pltpu.make_async_copy(row_hbm.at[pl.ds(_off(j), C)], rowi_v.at[ib],
                              semI[ib]).wait()
        pltpu.make_async_copy(ea_hbm.at[pl.ds(_off(j), C)], attr_v.at[ib % 2],
                              semI[ib]).wait()

    b_issue_idx(0, 0)
    b_issue_idx(1, 1)

    @pl.loop(0, (CPW - 1) // 4)
    def _ring_b(g):
        for b in range(4):
            j = 4 * g + b
            rb, ib, ipb = b % 2, b, (b + 2) % 4
            b_wait_idx(ib, j)
            if b < 2:
                @pl.when(g >= 1)
                def _():
                    a_wait_scatter(rb, ipb)
            else:
                a_wait_scatter(rb, ipb)

            @pl.loop(0, C)
            def _stage(k):
                rows_v[rb, k, pl.ds(0, 16)] = attr_v[rb, k, :]

            @pl.when(j + 2 <= CPW - 1)
            def _():
                b_issue_idx(ipb, j + 2)

            pltpu.async_copy(rows_v.at[rb], acc.at[rowi_v.at[ib]],
                             semS[rb], add=True)

    b_wait_idx(0, CPW - 1)
    a_wait_scatter(0, 2)
    @pl.loop(0, C)
    def _stage_last(k):
        rows_v[0, k, pl.ds(0, 16)] = attr_v[0, k, :]
    pltpu.async_copy(rows_v.at[0], acc.at[rowi_v.at[0]], semS[0], add=True)
    a_wait_scatter(0, 0)
    a_wait_scatter(1, 3)

    plsc.subcore_barrier()

    def _export_ec(r0, n, idx_ref):
        pltpu.async_copy(acc.at[idx_ref], rows_v.at[0, pl.ds(0, n)],
                         sem).wait()

        @pl.loop(0, n)
        def _stage_e(k):
            attr_v[0, k, :] = rows_v[0, k, pl.ds(0, 16)]

        pltpu.sync_copy(attr_v.at[0, pl.ds(0, n)], pe_hbm.at[cid, pl.ds(r0, n)])

        @pl.loop(0, n)
        def _stage_c(k):
            attr_v[0, k, :] = rows_v[0, k, pl.ds(16, 16)]

        pltpu.sync_copy(attr_v.at[0, pl.ds(0, n)], pc_hbm.at[cid, pl.ds(r0, n)])

    for j in range(13):
        _set_zidx(base_r + j * Z, clamp=False)
        _export_ec(base_r + j * Z, Z, zidx_v)

    @pl.when(s == NS - 1)
    def _export_tail_b():
        tidx_v[...] = NS * RPT + lax.iota(jnp.int32, TAIL)
        _export_ec(NS * RPT, TAIL, tidx_v)


_B = 2000


def _tc_body(px_ref, pe_ref, pc_ref, x4_ref, wm_ref, we_ref, wu_ref,
             bme_ref, bu_ref, out_ref, scale_smem):
    i = pl.program_id(0)
    sumx = px_ref[0] + px_ref[1]
    sume = pe_ref[0] + pe_ref[1]
    cnt = pc_ref[0, :, 0:1] + pc_ref[1, :, 0:1]
    acc = jnp.dot(sumx, wm_ref[...], preferred_element_type=jnp.float32)
    acc = acc + jnp.dot(sume, we_ref[...], preferred_element_type=jnp.float32)
    acc = acc + cnt * bme_ref[...]
    mean = acc / (cnt + 1e-8)
    upd = jnp.dot(mean, wu_ref[...], preferred_element_type=jnp.float32)
    upd = upd + bu_ref[...]
    norm = jnp.sqrt(jnp.sum(upd * upd, axis=1, keepdims=True) + 1e-8)
    proj = upd * jnp.minimum(1.0, (1.0 - 1e-5) / norm)

    def _cr(r):
        def dist(u, v):
            return jnp.sqrt(jnp.sum((u - v) ** 2) + 1e-8)
        a, b, cc, d = r[0:1, :], r[1:2, :], r[2:3, :], r[3:4, :]
        return (dist(a, cc) * dist(b, d)) / (dist(a, d) * dist(b, cc) + 1e-8)

    @pl.when(i == 0)
    def _():
        cr_init = _cr(x4_ref[...])
        cr_cur = _cr(proj[0:4, :])
        scale_smem[0] = cr_init / (cr_cur + 1e-8)

    out_ref[...] = proj * scale_smem[0]


def _tc_epilogue(px, pe, pc, x4, W_msg, W_edge, W_upd, bme, bu):
    return pl.pallas_call(
        _tc_body,
        out_shape=jax.ShapeDtypeStruct((N, D_OUT), jnp.float32),
        grid=(N // _B,),
        in_specs=[
            pl.BlockSpec((NC, _B, D_IN), lambda i: (0, i, 0)),
            pl.BlockSpec((NC, _B, D_EDGE), lambda i: (0, i, 0)),
            pl.BlockSpec((NC, _B, D_EDGE), lambda i: (0, i, 0)),
            pl.BlockSpec((4, D_IN), lambda i: (0, 0)),
            pl.BlockSpec((D_IN, D_OUT), lambda i: (0, 0)),
            pl.BlockSpec((D_EDGE, D_OUT), lambda i: (0, 0)),
            pl.BlockSpec((D_OUT, D_OUT), lambda i: (0, 0)),
            pl.BlockSpec((1, D_OUT), lambda i: (0, 0)),
            pl.BlockSpec((1, D_OUT), lambda i: (0, 0)),
        ],
        out_specs=pl.BlockSpec((_B, D_OUT), lambda i: (i, 0)),
        scratch_shapes=[pltpu.SMEM((1,), jnp.float32)],
    )(px, pe, pc, x4, W_msg, W_edge, W_upd, bme, bu)


def kernel(x, edge_index, edge_attr, W_msg, b_msg, W_edge, b_edge, W_upd, b_upd):
    row = edge_index[0]
    col = edge_index[1]
    zero = jnp.zeros((C, D_IN), jnp.float32)
    px, pe, pc = _sc_segment_sums(x, row, col, edge_attr, zero)
    x4 = x[0:4]
    bme = (b_msg + b_edge).reshape(1, D_OUT)
    bu = b_upd.reshape(1, D_OUT)
    return _tc_epilogue(px, pe, pc, x4, W_msg, W_edge, W_upd, bme, bu)


# speedup vs baseline: 6.2622x; 1.1263x over previous
import functools

import jax
import jax.numpy as jnp
from jax import lax
from jax.experimental import pallas as pl
from jax.experimental.pallas import tpu as pltpu
from jax.experimental.pallas import tpu_sc as plsc

N = 10000
E = 320000
D_IN = 128
D_OUT = 128
D_EDGE = 16

NC = 2
NS = 16
NW = NC * NS
C = 64
NCHUNKS = E // C
CPW = NCHUNKS // NW
XW = NCHUNKS - NW * CPW
LAST = CPW - 1
RPT = 624
TAIL = N - NS * RPT
Z = 48

_mesh = plsc.VectorSubcoreMesh(core_axis_name="c", subcore_axis_name="s")


@functools.partial(
    pl.kernel,
    out_type=(
        jax.ShapeDtypeStruct((NC, N, D_IN), jnp.float32),
        jax.ShapeDtypeStruct((NC, N, D_EDGE), jnp.float32),
        jax.ShapeDtypeStruct((NC, N, D_EDGE), jnp.float32),
    ),
    mesh=_mesh,
    scratch_types=[
        pltpu.VMEM_SHARED((N, D_IN), jnp.float32),
        pltpu.VMEM((3, C, D_IN), jnp.float32),
        pltpu.VMEM((2, C, D_EDGE), jnp.float32),
        pltpu.VMEM((6, C), jnp.int32),
        pltpu.VMEM((6, C), jnp.int32),
        pltpu.VMEM((Z,), jnp.int32),
        pltpu.VMEM((TAIL,), jnp.int32),
        [pltpu.SemaphoreType.DMA] * 6,
        [pltpu.SemaphoreType.DMA] * 3,
        [pltpu.SemaphoreType.DMA] * 3,
        pltpu.SemaphoreType.DMA,
    ],
)
def _sc_segment_sums(x_hbm, row_hbm, col_hbm, ea_hbm, zero_hbm,
                     px_hbm, pe_hbm, pc_hbm,
                     acc, rows_v, attr_v, rowi_v, coli_v, zidx_v, tidx_v,
                     semI, semG, semS, sem):
    cid = lax.axis_index("c")
    s = lax.axis_index("s")
    w = cid * NS + s
    base_r = s * RPT

    def _set_zidx(base, clamp):
        for k in range(Z // 16):
            v = base + k * 16 + lax.iota(jnp.int32, 16)
            if clamp:
                v = jnp.minimum(v, N - 1)
            zidx_v[pl.ds(16 * k, 16)] = v

    def _zero_acc():
        for j in range(14):
            _set_zidx(base_r + j * Z, clamp=True)
            pltpu.async_copy(rows_v.at[0, pl.ds(0, Z)], acc.at[zidx_v],
                             sem).wait()

    def _off(j):
        return (w + j * NW) * C

    def wait_scatter(rb, ib):
        pltpu.make_async_copy(rows_v.at[rb], acc.at[rowi_v.at[ib]],
                              semS[rb]).wait()

    def issue_scatter(rb, ib):
        pltpu.async_copy(rows_v.at[rb], acc.at[rowi_v.at[ib]],
                         semS[rb], add=True)

    pltpu.sync_copy(zero_hbm, rows_v.at[0])
    _zero_acc()
    plsc.subcore_barrier()

    def a_issue_idx(ib, j):
        pltpu.async_copy(row_hbm.at[pl.ds(_off(j), C)], rowi_v.at[ib],
                         semI[ib])
        pltpu.async_copy(col_hbm.at[pl.ds(_off(j), C)], coli_v.at[ib],
                         semI[ib])

    def a_wait_idx(ib, j):
        pltpu.make_async_copy(row_hbm.at[pl.ds(_off(j), C)], rowi_v.at[ib],
                              semI[ib]).wait()
        pltpu.make_async_copy(col_hbm.at[pl.ds(_off(j), C)], coli_v.at[ib],
                              semI[ib]).wait()

    def issue_gather(rb, ib):
        pltpu.async_copy(x_hbm.at[coli_v.at[ib]], rows_v.at[rb], semG[rb])

    def wait_gather(rb, ib):
        pltpu.make_async_copy(x_hbm.at[coli_v.at[ib]], rows_v.at[rb],
                              semG[rb]).wait()

    for ib in range(3):
        a_issue_idx(ib, ib)

    @pl.loop(0, CPW // 6)
    def _ring_a(g):
        for b in range(6):
            j = 6 * g + b
            rb, ib = b % 3, b
            pb, qb = (b - 1) % 3, (b - 2) % 3
            pib, qib = (b - 1) % 6, (b - 2) % 6
            sib = (b + 3) % 6
            a_wait_idx(ib, j)
            if b >= 3:
                wait_scatter(rb, sib)
            else:
                @pl.when(g >= 1)
                def _():
                    wait_scatter(rb, sib)

            @pl.when(j + 3 <= LAST)
            def _():
                a_issue_idx(sib, j + 3)

            issue_gather(rb, ib)
            if b >= 2:
                wait_gather(qb, qib)
                issue_scatter(qb, qib)
            else:
                @pl.when(g >= 1)
                def _():
                    wait_gather(qb, qib)
                    issue_scatter(qb, qib)

    wait_gather(1, 4)
    issue_scatter(1, 4)
    wait_gather(2, 5)
    issue_scatter(2, 5)
    wait_scatter(0, 3)

    @pl.when(w < XW)
    def _extra_a():
        a_issue_idx(0, CPW)
        a_wait_idx(0, CPW)
        issue_gather(0, 0)
        wait_gather(0, 0)
        issue_scatter(0, 0)
        wait_scatter(0, 0)

    wait_scatter(1, 4)
    wait_scatter(2, 5)

    plsc.subcore_barrier()

    for j in range(13):
        _set_zidx(base_r + j * Z, clamp=False)
        r0 = base_r + j * Z
        pltpu.async_copy(acc.at[zidx_v], rows_v.at[0, pl.ds(0, Z)],
                         sem).wait()
        pltpu.sync_copy(rows_v.at[0, pl.ds(0, Z)], px_hbm.at[cid, pl.ds(r0, Z)])

    @pl.when(s == NS - 1)
    def _export_tail_a():
        tidx_v[...] = NS * RPT + lax.iota(jnp.int32, TAIL)
        pltpu.async_copy(acc.at[tidx_v], rows_v.at[0, pl.ds(0, TAIL)],
                         sem).wait()
        pltpu.sync_copy(rows_v.at[0, pl.ds(0, TAIL)],
                        px_hbm.at[cid, pl.ds(NS * RPT, TAIL)])

    plsc.subcore_barrier()

    pltpu.sync_copy(zero_hbm, rows_v.at[0])
    _zero_acc()
    pltpu.sync_copy(zero_hbm, rows_v.at[1])
    pltpu.sync_copy(zero_hbm, rows_v.at[2])

    onehot16 = jnp.where(lax.iota(jnp.int32, 16) == 0,
                         jnp.float32(1), jnp.float32(0))

    @pl.loop(0, C)
    def _ones_col(i):
        rows_v[0, i, pl.ds(16, 16)] = onehot16
        rows_v[1, i, pl.ds(16, 16)] = onehot16
        rows_v[2, i, pl.ds(16, 16)] = onehot16

    plsc.subcore_barrier()

    def b_issue_idx(ib, j):
        pltpu.async_copy(row_hbm.at[pl.ds(_off(j), C)], rowi_v.at[ib],
                         semI[ib])
        pltpu.async_copy(ea_hbm.at[pl.ds(_off(j), C)], attr_v.at[ib % 2],
                         semI[ib])

    def b_wait_idx(ib, j):
        pltpu.make_async_copy(row_hbm.at[pl.ds(_off(j), C)], rowi_v.at[ib],
                              semI[ib]).wait()
        pltpu.make_async_copy(ea_hbm.at[pl.ds(_off(j), C)], attr_v.at[ib % 2],
                              semI[ib]).wait()

    def b_stage(rb, ab):
        @pl.loop(0, C)
        def _stage(k):
            rows_v[rb, k, pl.ds(0, 16)] = attr_v[ab, k, :]

    for ib in range(2):
        b_issue_idx(ib, ib)

    @pl.loop(0, CPW // 6)
    def _ring_b(g):
        for b in range(6):
            j = 6 * g + b
            rb, ib = b % 3, b
            sib = (b + 3) % 6
            nib = (b + 2) % 6
            b_wait_idx(ib, j)
            if b >= 3:
                wait_scatter(rb, sib)
            else:
                @pl.when(g >= 1)
                def _():
                    wait_scatter(rb, sib)

            b_stage(rb, b % 2)

            @pl.when(j + 2 <= LAST)
            def _():
                b_issue_idx(nib, j + 2)

            issue_scatter(rb, ib)

    wait_scatter(0, 3)
    wait_scatter(1, 4)
    wait_scatter(2, 5)

    @pl.when(w < XW)
    def _extra_b():
        b_issue_idx(0, CPW)
        b_wait_idx(0, CPW)
        b_stage(0, 0)
        issue_scatter(0, 0)
        wait_scatter(0, 0)

    plsc.subcore_barrier()

    def _export_ec(r0, n, idx_ref):
        pltpu.async_copy(acc.at[idx_ref], rows_v.at[0, pl.ds(0, n)],
                         sem).wait()

        @pl.loop(0, n)
        def _stage_e(k):
            attr_v[0, k, :] = rows_v[0, k, pl.ds(0, 16)]

        pltpu.sync_copy(attr_v.at[0, pl.ds(0, n)], pe_hbm.at[cid, pl.ds(r0, n)])

        @pl.loop(0, n)
        def _stage_c(k):
            attr_v[0, k, :] = rows_v[0, k, pl.ds(16, 16)]

        pltpu.sync_copy(attr_v.at[0, pl.ds(0, n)], pc_hbm.at[cid, pl.ds(r0, n)])

    for j in range(13):
        _set_zidx(base_r + j * Z, clamp=False)
        _export_ec(base_r + j * Z, Z, zidx_v)

    @pl.when(s == NS - 1)
    def _export_tail_b():
        tidx_v[...] = NS * RPT + lax.iota(jnp.int32, TAIL)
        _export_ec(NS * RPT, TAIL, tidx_v)


_B = 2000


def _tc_body(px_ref, pe_ref, pc_ref, x4_ref, wm_ref, we_ref, wu_ref,
             bme_ref, bu_ref, out_ref, scale_smem):
    i = pl.program_id(0)
    sumx = px_ref[0] + px_ref[1]
    sume = pe_ref[0] + pe_ref[1]
    cnt = pc_ref[0, :, 0:1] + pc_ref[1, :, 0:1]
    acc = jnp.dot(sumx, wm_ref[...], preferred_element_type=jnp.float32)
    acc = acc + jnp.dot(sume, we_ref[...], preferred_element_type=jnp.float32)
    acc = acc + cnt * bme_ref[...]
    mean = acc / (cnt + 1e-8)
    upd = jnp.dot(mean, wu_ref[...], preferred_element_type=jnp.float32)
    upd = upd + bu_ref[...]
    norm = jnp.sqrt(jnp.sum(upd * upd, axis=1, keepdims=True) + 1e-8)
    proj = upd * jnp.minimum(1.0, (1.0 - 1e-5) / norm)

    def _cr(r):
        def dist(u, v):
            return jnp.sqrt(jnp.sum((u - v) ** 2) + 1e-8)
        a, b, cc, d = r[0:1, :], r[1:2, :], r[2:3, :], r[3:4, :]
        return (dist(a, cc) * dist(b, d)) / (dist(a, d) * dist(b, cc) + 1e-8)

    @pl.when(i == 0)
    def _():
        cr_init = _cr(x4_ref[...])
        cr_cur = _cr(proj[0:4, :])
        scale_smem[0] = cr_init / (cr_cur + 1e-8)

    out_ref[...] = proj * scale_smem[0]


def _tc_epilogue(px, pe, pc, x4, W_msg, W_edge, W_upd, bme, bu):
    return pl.pallas_call(
        _tc_body,
        out_shape=jax.ShapeDtypeStruct((N, D_OUT), jnp.float32),
        grid=(N // _B,),
        in_specs=[
            pl.BlockSpec((NC, _B, D_IN), lambda i: (0, i, 0)),
            pl.BlockSpec((NC, _B, D_EDGE), lambda i: (0, i, 0)),
            pl.BlockSpec((NC, _B, D_EDGE), lambda i: (0, i, 0)),
            pl.BlockSpec((4, D_IN), lambda i: (0, 0)),
            pl.BlockSpec((D_IN, D_OUT), lambda i: (0, 0)),
            pl.BlockSpec((D_EDGE, D_OUT), lambda i: (0, 0)),
            pl.BlockSpec((D_OUT, D_OUT), lambda i: (0, 0)),
            pl.BlockSpec((1, D_OUT), lambda i: (0, 0)),
            pl.BlockSpec((1, D_OUT), lambda i: (0, 0)),
        ],
        out_specs=pl.BlockSpec((_B, D_OUT), lambda i: (i, 0)),
        scratch_shapes=[pltpu.SMEM((1,), jnp.float32)],
    )(px, pe, pc, x4, W_msg, W_edge, W_upd, bme, bu)


def kernel(x, edge_index, edge_attr, W_msg, b_msg, W_edge, b_edge, W_upd, b_upd):
    row = edge_index[0]
    col = edge_index[1]
    zero = jnp.zeros((C, D_IN), jnp.float32)
    px, pe, pc = _sc_segment_sums(x, row, col, edge_attr, zero)
    x4 = x[0:4]
    bme = (b_msg + b_edge).reshape(1, D_OUT)
    bu = b_upd.reshape(1, D_OUT)
    return _tc_epilogue(px, pe, pc, x4, W_msg, W_edge, W_upd, bme, bu)


# speedup vs baseline: 6.4250x; 1.0260x over previous
import functools

import jax
import jax.numpy as jnp
from jax import lax
from jax.experimental import pallas as pl
from jax.experimental.pallas import tpu as pltpu
from jax.experimental.pallas import tpu_sc as plsc

N = 10000
E = 320000
D_IN = 128
D_OUT = 128
D_EDGE = 16

NC = 2
NS = 16
NW = NC * NS
C = 64
NCHUNKS = E // C
CPW = NCHUNKS // NW
XW = NCHUNKS - NW * CPW
LAST = CPW - 1
RPT = 624
TAIL = N - NS * RPT
Z = 48

_mesh = plsc.VectorSubcoreMesh(core_axis_name="c", subcore_axis_name="s")


@functools.partial(
    pl.kernel,
    out_type=(
        jax.ShapeDtypeStruct((NC, N, D_IN), jnp.float32),
        jax.ShapeDtypeStruct((NC, N, D_EDGE), jnp.float32),
        jax.ShapeDtypeStruct((NC, N, D_EDGE), jnp.float32),
    ),
    mesh=_mesh,
    scratch_types=[
        pltpu.VMEM_SHARED((N, D_IN), jnp.float32),
        pltpu.VMEM((3, C, D_IN), jnp.float32),
        pltpu.VMEM((2, C, D_EDGE), jnp.float32),
        pltpu.VMEM((6, C), jnp.int32),
        pltpu.VMEM((6, C), jnp.int32),
        pltpu.VMEM((3, Z), jnp.int32),
        pltpu.VMEM((TAIL,), jnp.int32),
        [pltpu.SemaphoreType.DMA] * 6,
        [pltpu.SemaphoreType.DMA] * 3,
        [pltpu.SemaphoreType.DMA] * 3,
        pltpu.SemaphoreType.DMA,
    ],
)
def _sc_segment_sums(x_hbm, row_hbm, col_hbm, ea_hbm, zero_hbm,
                     px_hbm, pe_hbm, pc_hbm,
                     acc, rows_v, attr_v, rowi_v, coli_v, zidx_v, tidx_v,
                     semI, semG, semS, sem):
    cid = lax.axis_index("c")
    s = lax.axis_index("s")
    w = cid * NS + s
    base_r = s * RPT

    def _set_zidx(zb, base, clamp):
        for k in range(Z // 16):
            v = base + k * 16 + lax.iota(jnp.int32, 16)
            if clamp:
                v = jnp.minimum(v, N - 1)
            zidx_v[zb, pl.ds(16 * k, 16)] = v

    def _zero_acc():
        for j in range(14):
            zb = j % 3
            if j >= 3:
                pltpu.make_async_copy(rows_v.at[0, pl.ds(0, Z)],
                                      acc.at[zidx_v.at[zb]], semS[zb]).wait()
            _set_zidx(zb, base_r + j * Z, clamp=True)
            pltpu.async_copy(rows_v.at[0, pl.ds(0, Z)], acc.at[zidx_v.at[zb]],
                             semS[zb])
        for j in range(11, 14):
            zb = j % 3
            pltpu.make_async_copy(rows_v.at[0, pl.ds(0, Z)],
                                  acc.at[zidx_v.at[zb]], semS[zb]).wait()

    def _off(j):
        return (w + j * NW) * C

    def wait_scatter(rb, ib):
        pltpu.make_async_copy(rows_v.at[rb], acc.at[rowi_v.at[ib]],
                              semS[rb]).wait()

    def issue_scatter(rb, ib):
        pltpu.async_copy(rows_v.at[rb], acc.at[rowi_v.at[ib]],
                         semS[rb], add=True)

    pltpu.sync_copy(zero_hbm, rows_v.at[0])
    _zero_acc()
    plsc.subcore_barrier()

    def a_issue_idx(ib, j):
        pltpu.async_copy(row_hbm.at[pl.ds(_off(j), C)], rowi_v.at[ib],
                         semI[ib])
        pltpu.async_copy(col_hbm.at[pl.ds(_off(j), C)], coli_v.at[ib],
                         semI[ib])

    def a_wait_idx(ib, j):
        pltpu.make_async_copy(row_hbm.at[pl.ds(_off(j), C)], rowi_v.at[ib],
                              semI[ib]).wait()
        pltpu.make_async_copy(col_hbm.at[pl.ds(_off(j), C)], coli_v.at[ib],
                              semI[ib]).wait()

    def issue_gather(rb, ib):
        pltpu.async_copy(x_hbm.at[coli_v.at[ib]], rows_v.at[rb], semG[rb])

    def wait_gather(rb, ib):
        pltpu.make_async_copy(x_hbm.at[coli_v.at[ib]], rows_v.at[rb],
                              semG[rb]).wait()

    for ib in range(3):
        a_issue_idx(ib, ib)

    @pl.loop(0, CPW // 6)
    def _ring_a(g):
        for b in range(6):
            j = 6 * g + b
            rb, ib = b % 3, b
            pb, qb = (b - 1) % 3, (b - 2) % 3
            pib, qib = (b - 1) % 6, (b - 2) % 6
            sib = (b + 3) % 6
            a_wait_idx(ib, j)
            if b >= 3:
                wait_scatter(rb, sib)
            else:
                @pl.when(g >= 1)
                def _():
                    wait_scatter(rb, sib)

            @pl.when(j + 3 <= LAST)
            def _():
                a_issue_idx(sib, j + 3)

            issue_gather(rb, ib)
            if b >= 2:
                wait_gather(qb, qib)
                issue_scatter(qb, qib)
            else:
                @pl.when(g >= 1)
                def _():
                    wait_gather(qb, qib)
                    issue_scatter(qb, qib)

    wait_gather(1, 4)
    issue_scatter(1, 4)
    wait_gather(2, 5)
    issue_scatter(2, 5)
    wait_scatter(0, 3)

    @pl.when(w < XW)
    def _extra_a():
        a_issue_idx(0, CPW)
        a_wait_idx(0, CPW)
        issue_gather(0, 0)
        wait_gather(0, 0)
        issue_scatter(0, 0)
        wait_scatter(0, 0)

    wait_scatter(1, 4)
    wait_scatter(2, 5)

    plsc.subcore_barrier()

    for j in range(14):
        zb = j % 3
        if j < 13:
            _set_zidx(zb, base_r + j * Z, clamp=False)
            pltpu.async_copy(acc.at[zidx_v.at[zb]], rows_v.at[zb, pl.ds(0, Z)],
                             semG[zb])
        if j >= 1:
            pb = (j - 1) % 3
            pltpu.make_async_copy(acc.at[zidx_v.at[pb]],
                                  rows_v.at[pb, pl.ds(0, Z)], semG[pb]).wait()
            pltpu.sync_copy(rows_v.at[pb, pl.ds(0, Z)],
                            px_hbm.at[cid, pl.ds(base_r + (j - 1) * Z, Z)])

    @pl.when(s == NS - 1)
    def _export_tail_a():
        tidx_v[...] = NS * RPT + lax.iota(jnp.int32, TAIL)
        pltpu.async_copy(acc.at[tidx_v], rows_v.at[0, pl.ds(0, TAIL)],
                         sem).wait()
        pltpu.sync_copy(rows_v.at[0, pl.ds(0, TAIL)],
                        px_hbm.at[cid, pl.ds(NS * RPT, TAIL)])

    plsc.subcore_barrier()

    pltpu.sync_copy(zero_hbm, rows_v.at[0])
    _zero_acc()
    pltpu.sync_copy(zero_hbm, rows_v.at[1])
    pltpu.sync_copy(zero_hbm, rows_v.at[2])

    onehot16 = jnp.where(lax.iota(jnp.int32, 16) == 0,
                         jnp.float32(1), jnp.float32(0))

    @pl.loop(0, C)
    def _ones_col(i):
        rows_v[0, i, pl.ds(16, 16)] = onehot16
        rows_v[1, i, pl.ds(16, 16)] = onehot16
        rows_v[2, i, pl.ds(16, 16)] = onehot16

    plsc.subcore_barrier()

    def b_issue_idx(ib, j):
        pltpu.async_copy(row_hbm.at[pl.ds(_off(j), C)], rowi_v.at[ib],
                         semI[ib])
        pltpu.async_copy(ea_hbm.at[pl.ds(_off(j), C)], attr_v.at[ib % 2],
                         semI[ib])

    def b_wait_idx(ib, j):
        pltpu.make_async_copy(row_hbm.at[pl.ds(_off(j), C)], rowi_v.at[ib],
                              semI[ib]).wait()
        pltpu.make_async_copy(ea_hbm.at[pl.ds(_off(j), C)], attr_v.at[ib % 2],
                              semI[ib]).wait()

    def b_stage(rb, ab):
        @pl.loop(0, C)
        def _stage(k):
            rows_v[rb, k, pl.ds(0, 16)] = attr_v[ab, k, :]

    for ib in range(2):
        b_issue_idx(ib, ib)

    @pl.loop(0, CPW // 6)
    def _ring_b(g):
        for b in range(6):
            j = 6 * g + b
            rb, ib = b % 3, b
            sib = (b + 3) % 6
            nib = (b + 2) % 6
            b_wait_idx(ib, j)
            if b >= 3:
                wait_scatter(rb, sib)
            else:
                @pl.when(g >= 1)
                def _():
                    wait_scatter(rb, sib)

            b_stage(rb, b % 2)

            @pl.when(j + 2 <= LAST)
            def _():
                b_issue_idx(nib, j + 2)

            issue_scatter(rb, ib)

    wait_scatter(0, 3)
    wait_scatter(1, 4)
    wait_scatter(2, 5)

    @pl.when(w < XW)
    def _extra_b():
        b_issue_idx(0, CPW)
        b_wait_idx(0, CPW)
        b_stage(0, 0)
        issue_scatter(0, 0)
        wait_scatter(0, 0)

    plsc.subcore_barrier()

    def _export_ec(r0, n, idx_ref):
        pltpu.async_copy(acc.at[idx_ref], rows_v.at[0, pl.ds(0, n)],
                         sem).wait()

        @pl.loop(0, n)
        def _stage_e(k):
            attr_v[0, k, :] = rows_v[0, k, pl.ds(0, 16)]

        pltpu.sync_copy(attr_v.at[0, pl.ds(0, n)], pe_hbm.at[cid, pl.ds(r0, n)])

        @pl.loop(0, n)
        def _stage_c(k):
            attr_v[0, k, :] = rows_v[0, k, pl.ds(16, 16)]

        pltpu.sync_copy(attr_v.at[0, pl.ds(0, n)], pc_hbm.at[cid, pl.ds(r0, n)])

    for j in range(14):
        zb = j % 3
        if j < 13:
            _set_zidx(zb, base_r + j * Z, clamp=False)
            pltpu.async_copy(acc.at[zidx_v.at[zb]], rows_v.at[zb, pl.ds(0, Z)],
                             semG[zb])
        if j >= 1:
            pb = (j - 1) % 3
            r0 = base_r + (j - 1) * Z
            pltpu.make_async_copy(acc.at[zidx_v.at[pb]],
                                  rows_v.at[pb, pl.ds(0, Z)], semG[pb]).wait()
            if j >= 2:
                q0 = base_r + (j - 2) * Z
                pltpu.make_async_copy(attr_v.at[0, pl.ds(0, Z)],
                                      pe_hbm.at[cid, pl.ds(q0, Z)],
                                      semS[0]).wait()
                pltpu.make_async_copy(attr_v.at[1, pl.ds(0, Z)],
                                      pc_hbm.at[cid, pl.ds(q0, Z)],
                                      semS[1]).wait()

            @pl.loop(0, Z)
            def _stage_e(k):
                attr_v[0, k, :] = rows_v[pb, k, pl.ds(0, 16)]

            @pl.loop(0, Z)
            def _stage_c(k):
                attr_v[1, k, :] = rows_v[pb, k, pl.ds(16, 16)]

            pltpu.async_copy(attr_v.at[0, pl.ds(0, Z)],
                             pe_hbm.at[cid, pl.ds(r0, Z)], semS[0])
            pltpu.async_copy(attr_v.at[1, pl.ds(0, Z)],
                             pc_hbm.at[cid, pl.ds(r0, Z)], semS[1])
    pltpu.make_async_copy(attr_v.at[0, pl.ds(0, Z)],
                          pe_hbm.at[cid, pl.ds(base_r + 12 * Z, Z)],
                          semS[0]).wait()
    pltpu.make_async_copy(attr_v.at[1, pl.ds(0, Z)],
                          pc_hbm.at[cid, pl.ds(base_r + 12 * Z, Z)],
                          semS[1]).wait()

    @pl.when(s == NS - 1)
    def _export_tail_b():
        tidx_v[...] = NS * RPT + lax.iota(jnp.int32, TAIL)
        _export_ec(NS * RPT, TAIL, tidx_v)


_B = 2000


def _tc_body(px_ref, pe_ref, pc_ref, x4_ref, wm_ref, we_ref, wu_ref,
             bme_ref, bu_ref, out_ref, scale_smem):
    i = pl.program_id(0)
    sumx = px_ref[0] + px_ref[1]
    sume = pe_ref[0] + pe_ref[1]
    cnt = pc_ref[0, :, 0:1] + pc_ref[1, :, 0:1]
    acc = jnp.dot(sumx, wm_ref[...], preferred_element_type=jnp.float32)
    acc = acc + jnp.dot(sume, we_ref[...], preferred_element_type=jnp.float32)
    acc = acc + cnt * bme_ref[...]
    mean = acc / (cnt + 1e-8)
    upd = jnp.dot(mean, wu_ref[...], preferred_element_type=jnp.float32)
    upd = upd + bu_ref[...]
    norm = jnp.sqrt(jnp.sum(upd * upd, axis=1, keepdims=True) + 1e-8)
    proj = upd * jnp.minimum(1.0, (1.0 - 1e-5) / norm)

    def _cr(r):
        def dist(u, v):
            return jnp.sqrt(jnp.sum((u - v) ** 2) + 1e-8)
        a, b, cc, d = r[0:1, :], r[1:2, :], r[2:3, :], r[3:4, :]
        return (dist(a, cc) * dist(b, d)) / (dist(a, d) * dist(b, cc) + 1e-8)

    @pl.when(i == 0)
    def _():
        cr_init = _cr(x4_ref[...])
        cr_cur = _cr(proj[0:4, :])
        scale_smem[0] = cr_init / (cr_cur + 1e-8)

    out_ref[...] = proj * scale_smem[0]


def _tc_epilogue(px, pe, pc, x4, W_msg, W_edge, W_upd, bme, bu):
    return pl.pallas_call(
        _tc_body,
        out_shape=jax.ShapeDtypeStruct((N, D_OUT), jnp.float32),
        grid=(N // _B,),
        in_specs=[
            pl.BlockSpec((NC, _B, D_IN), lambda i: (0, i, 0)),
            pl.BlockSpec((NC, _B, D_EDGE), lambda i: (0, i, 0)),
            pl.BlockSpec((NC, _B, D_EDGE), lambda i: (0, i, 0)),
            pl.BlockSpec((4, D_IN), lambda i: (0, 0)),
            pl.BlockSpec((D_IN, D_OUT), lambda i: (0, 0)),
            pl.BlockSpec((D_EDGE, D_OUT), lambda i: (0, 0)),
            pl.BlockSpec((D_OUT, D_OUT), lambda i: (0, 0)),
            pl.BlockSpec((1, D_OUT), lambda i: (0, 0)),
            pl.BlockSpec((1, D_OUT), lambda i: (0, 0)),
        ],
        out_specs=pl.BlockSpec((_B, D_OUT), lambda i: (i, 0)),
        scratch_shapes=[pltpu.SMEM((1,), jnp.float32)],
    )(px, pe, pc, x4, W_msg, W_edge, W_upd, bme, bu)


def kernel(x, edge_index, edge_attr, W_msg, b_msg, W_edge, b_edge, W_upd, b_upd):
    row = edge_index[0]
    col = edge_index[1]
    zero = jnp.zeros((C, D_IN), jnp.float32)
    px, pe, pc = _sc_segment_sums(x, row, col, edge_attr, zero)
    x4 = x[0:4]
    bme = (b_msg + b_edge).reshape(1, D_OUT)
    bu = b_upd.reshape(1, D_OUT)
    return _tc_epilogue(px, pe, pc, x4, W_msg, W_edge, W_upd, bme, bu)


# speedup vs baseline: 6.5483x; 1.0192x over previous
import functools

import jax
import jax.numpy as jnp
from jax import lax
from jax.experimental import pallas as pl
from jax.experimental.pallas import tpu as pltpu
from jax.experimental.pallas import tpu_sc as plsc

N = 10000
E = 320000
D_IN = 128
D_OUT = 128
D_EDGE = 16

NC = 2
NS = 16
NW = NC * NS
C = 64
NCHUNKS = E // C
CPW = NCHUNKS // NW
XW = NCHUNKS - NW * CPW
LAST = CPW - 1
RPT = 624
TAIL = N - NS * RPT
Z = 48

_mesh = plsc.VectorSubcoreMesh(core_axis_name="c", subcore_axis_name="s")


@functools.partial(
    pl.kernel,
    out_type=(
        jax.ShapeDtypeStruct((NC, N, D_IN), jnp.float32),
        jax.ShapeDtypeStruct((NC, N, D_EDGE), jnp.float32),
        jax.ShapeDtypeStruct((NC, N, D_EDGE), jnp.float32),
    ),
    mesh=_mesh,
    scratch_types=[
        pltpu.VMEM_SHARED((N, D_IN), jnp.float32),
        pltpu.VMEM((3, C, D_IN), jnp.float32),
        pltpu.VMEM((2, C, D_EDGE), jnp.float32),
        pltpu.VMEM((6, C), jnp.int32),
        pltpu.VMEM((6, C), jnp.int32),
        pltpu.VMEM((3, Z), jnp.int32),
        pltpu.VMEM((TAIL,), jnp.int32),
        [pltpu.SemaphoreType.DMA] * 6,
        [pltpu.SemaphoreType.DMA] * 3,
        [pltpu.SemaphoreType.DMA] * 3,
        pltpu.SemaphoreType.DMA,
    ],
)
def _sc_segment_sums(x_hbm, row_hbm, col_hbm, ea_hbm, zero_hbm,
                     px_hbm, pe_hbm, pc_hbm,
                     acc, rows_v, attr_v, rowi_v, coli_v, zidx_v, tidx_v,
                     semI, semG, semS, sem):
    cid = lax.axis_index("c")
    s = lax.axis_index("s")
    w = cid * NS + s
    base_r = s * RPT

    def _set_zidx(zb, base, clamp):
        for k in range(Z // 16):
            v = base + k * 16 + lax.iota(jnp.int32, 16)
            if clamp:
                v = jnp.minimum(v, N - 1)
            zidx_v[zb, pl.ds(16 * k, 16)] = v

    def _zero_acc():
        for j in range(14):
            zb = j % 3
            if j >= 3:
                pltpu.make_async_copy(rows_v.at[0, pl.ds(0, Z)],
                                      acc.at[zidx_v.at[zb]], semS[zb]).wait()
            _set_zidx(zb, base_r + j * Z, clamp=True)
            pltpu.async_copy(rows_v.at[0, pl.ds(0, Z)], acc.at[zidx_v.at[zb]],
                             semS[zb])
        for j in range(11, 14):
            zb = j % 3
            pltpu.make_async_copy(rows_v.at[0, pl.ds(0, Z)],
                                  acc.at[zidx_v.at[zb]], semS[zb]).wait()

    def _off(j):
        return (w + j * NW) * C

    def wait_scatter(rb, ib):
        pltpu.make_async_copy(rows_v.at[rb], acc.at[rowi_v.at[ib]],
                              semS[rb]).wait()

    def issue_scatter(rb, ib):
        pltpu.async_copy(rows_v.at[rb], acc.at[rowi_v.at[ib]],
                         semS[rb], add=True)

    pltpu.sync_copy(zero_hbm, rows_v.at[0])
    _zero_acc()
    plsc.subcore_barrier()

    def a_issue_idx(ib, j):
        pltpu.async_copy(row_hbm.at[pl.ds(_off(j), C)], rowi_v.at[ib],
                         semI[ib])
        pltpu.async_copy(col_hbm.at[pl.ds(_off(j), C)], coli_v.at[ib],
                         semI[ib])

    def a_wait_idx(ib, j):
        pltpu.make_async_copy(row_hbm.at[pl.ds(_off(j), C)], rowi_v.at[ib],
                              semI[ib]).wait()
        pltpu.make_async_copy(col_hbm.at[pl.ds(_off(j), C)], coli_v.at[ib],
                              semI[ib]).wait()

    def issue_gather(rb, ib):
        pltpu.async_copy(x_hbm.at[coli_v.at[ib]], rows_v.at[rb], semG[rb])

    def wait_gather(rb, ib):
        pltpu.make_async_copy(x_hbm.at[coli_v.at[ib]], rows_v.at[rb],
                              semG[rb]).wait()

    for ib in range(3):
        a_issue_idx(ib, ib)

    @pl.loop(0, CPW // 6)
    def _ring_a(g):
        for b in range(6):
            j = 6 * g + b
            rb, ib = b % 3, b
            pb, qb = (b - 1) % 3, (b - 2) % 3
            pib, qib = (b - 1) % 6, (b - 2) % 6
            sib = (b + 3) % 6
            a_wait_idx(ib, j)
            if b >= 3:
                wait_scatter(rb, sib)
            else:
                @pl.when(g >= 1)
                def _():
                    wait_scatter(rb, sib)

            @pl.when(j + 3 <= LAST)
            def _():
                a_issue_idx(sib, j + 3)

            issue_gather(rb, ib)
            if b >= 2:
                wait_gather(qb, qib)
                issue_scatter(qb, qib)
            else:
                @pl.when(g >= 1)
                def _():
                    wait_gather(qb, qib)
                    issue_scatter(qb, qib)

    wait_gather(1, 4)
    issue_scatter(1, 4)
    wait_gather(2, 5)
    issue_scatter(2, 5)
    wait_scatter(0, 3)

    @pl.when(w < XW)
    def _extra_a():
        a_issue_idx(0, CPW)
        a_wait_idx(0, CPW)
        issue_gather(0, 0)
        wait_gather(0, 0)
        issue_scatter(0, 0)
        wait_scatter(0, 0)

    wait_scatter(1, 4)
    wait_scatter(2, 5)

    plsc.subcore_barrier()

    for j in range(14):
        zb = j % 3
        if j < 13:
            _set_zidx(zb, base_r + j * Z, clamp=False)
            pltpu.async_copy(acc.at[zidx_v.at[zb]], rows_v.at[zb, pl.ds(0, Z)],
                             semG[zb])
        if j >= 1:
            pb = (j - 1) % 3
            pltpu.make_async_copy(acc.at[zidx_v.at[pb]],
                                  rows_v.at[pb, pl.ds(0, Z)], semG[pb]).wait()
            pltpu.sync_copy(rows_v.at[pb, pl.ds(0, Z)],
                            px_hbm.at[cid, pl.ds(base_r + (j - 1) * Z, Z)])

    @pl.when(s == NS - 1)
    def _export_tail_a():
        tidx_v[...] = NS * RPT + lax.iota(jnp.int32, TAIL)
        pltpu.async_copy(acc.at[tidx_v], rows_v.at[0, pl.ds(0, TAIL)],
                         sem).wait()
        pltpu.sync_copy(rows_v.at[0, pl.ds(0, TAIL)],
                        px_hbm.at[cid, pl.ds(NS * RPT, TAIL)])

    plsc.subcore_barrier()

    pltpu.sync_copy(zero_hbm, rows_v.at[0])
    _zero_acc()
    pltpu.sync_copy(zero_hbm, rows_v.at[1])
    pltpu.sync_copy(zero_hbm, rows_v.at[2])

    onehot16 = jnp.where(lax.iota(jnp.int32, 16) == 0,
                         jnp.float32(1), jnp.float32(0))

    @pl.loop(0, C)
    def _ones_col(i):
        rows_v[0, i, pl.ds(16, 16)] = onehot16
        rows_v[1, i, pl.ds(16, 16)] = onehot16
        rows_v[2, i, pl.ds(16, 16)] = onehot16

    plsc.subcore_barrier()

    def b_issue_idx(ib, j):
        pltpu.async_copy(row_hbm.at[pl.ds(_off(j), C)], rowi_v.at[ib],
                         semI[ib])
        pltpu.async_copy(ea_hbm.at[pl.ds(_off(j), C)], attr_v.at[ib % 2],
                         semI[ib])

    def b_wait_idx(ib, j):
        pltpu.make_async_copy(row_hbm.at[pl.ds(_off(j), C)], rowi_v.at[ib],
                              semI[ib]).wait()
        pltpu.make_async_copy(ea_hbm.at[pl.ds(_off(j), C)], attr_v.at[ib % 2],
                              semI[ib]).wait()

    def b_stage(rb, ab):
        @pl.loop(0, C, unroll=8)
        def _stage(k):
            rows_v[rb, k, pl.ds(0, 16)] = attr_v[ab, k, :]

    for ib in range(2):
        b_issue_idx(ib, ib)

    @pl.loop(0, CPW // 6)
    def _ring_b(g):
        for b in range(6):
            j = 6 * g + b
            rb, ib = b % 3, b
            sib = (b + 3) % 6
            nib = (b + 2) % 6
            b_wait_idx(ib, j)
            if b >= 3:
                wait_scatter(rb, sib)
            else:
                @pl.when(g >= 1)
                def _():
                    wait_scatter(rb, sib)

            b_stage(rb, b % 2)

            @pl.when(j + 2 <= LAST)
            def _():
                b_issue_idx(nib, j + 2)

            issue_scatter(rb, ib)

    wait_scatter(0, 3)
    wait_scatter(1, 4)
    wait_scatter(2, 5)

    @pl.when(w < XW)
    def _extra_b():
        b_issue_idx(0, CPW)
        b_wait_idx(0, CPW)
        b_stage(0, 0)
        issue_scatter(0, 0)
        wait_scatter(0, 0)

    plsc.subcore_barrier()

    def _export_ec(r0, n, idx_ref):
        pltpu.async_copy(acc.at[idx_ref], rows_v.at[0, pl.ds(0, n)],
                         sem).wait()

        @pl.loop(0, n)
        def _stage_e(k):
            attr_v[0, k, :] = rows_v[0, k, pl.ds(0, 16)]

        pltpu.sync_copy(attr_v.at[0, pl.ds(0, n)], pe_hbm.at[cid, pl.ds(r0, n)])

        @pl.loop(0, n)
        def _stage_c(k):
            attr_v[0, k, :] = rows_v[0, k, pl.ds(16, 16)]

        pltpu.sync_copy(attr_v.at[0, pl.ds(0, n)], pc_hbm.at[cid, pl.ds(r0, n)])

    for j in range(14):
        zb = j % 3
        if j < 13:
            _set_zidx(zb, base_r + j * Z, clamp=False)
            pltpu.async_copy(acc.at[zidx_v.at[zb]], rows_v.at[zb, pl.ds(0, Z)],
                             semG[zb])
        if j >= 1:
            pb = (j - 1) % 3
            r0 = base_r + (j - 1) * Z
            pltpu.make_async_copy(acc.at[zidx_v.at[pb]],
                                  rows_v.at[pb, pl.ds(0, Z)], semG[pb]).wait()
            if j >= 2:
                q0 = base_r + (j - 2) * Z
                pltpu.make_async_copy(attr_v.at[0, pl.ds(0, Z)],
                                      pe_hbm.at[cid, pl.ds(q0, Z)],
                                      semS[0]).wait()
                pltpu.make_async_copy(attr_v.at[1, pl.ds(0, Z)],
                                      pc_hbm.at[cid, pl.ds(q0, Z)],
                                      semS[1]).wait()

            @pl.loop(0, Z, unroll=8)
            def _stage_e(k):
                attr_v[0, k, :] = rows_v[pb, k, pl.ds(0, 16)]

            @pl.loop(0, Z, unroll=8)
            def _stage_c(k):
                attr_v[1, k, :] = rows_v[pb, k, pl.ds(16, 16)]

            pltpu.async_copy(attr_v.at[0, pl.ds(0, Z)],
                             pe_hbm.at[cid, pl.ds(r0, Z)], semS[0])
            pltpu.async_copy(attr_v.at[1, pl.ds(0, Z)],
                             pc_hbm.at[cid, pl.ds(r0, Z)], semS[1])
    pltpu.make_async_copy(attr_v.at[0, pl.ds(0, Z)],
                          pe_hbm.at[cid, pl.ds(base_r + 12 * Z, Z)],
                          semS[0]).wait()
    pltpu.make_async_copy(attr_v.at[1, pl.ds(0, Z)],
                          pc_hbm.at[cid, pl.ds(base_r + 12 * Z, Z)],
                          semS[1]).wait()

    @pl.when(s == NS - 1)
    def _export_tail_b():
        tidx_v[...] = NS * RPT + lax.iota(jnp.int32, TAIL)
        _export_ec(NS * RPT, TAIL, tidx_v)


_B = 2000


def _tc_body(px_ref, pe_ref, pc_ref, x4_ref, wm_ref, we_ref, wu_ref,
             bme_ref, bu_ref, out_ref, scale_smem):
    i = pl.program_id(0)
    sumx = px_ref[0] + px_ref[1]
    sume = pe_ref[0] + pe_ref[1]
    cnt = pc_ref[0, :, 0:1] + pc_ref[1, :, 0:1]
    acc = jnp.dot(sumx, wm_ref[...], preferred_element_type=jnp.float32)
    acc = acc + jnp.dot(sume, we_ref[...], preferred_element_type=jnp.float32)
    acc = acc + cnt * bme_ref[...]
    mean = acc / (cnt + 1e-8)
    upd = jnp.dot(mean, wu_ref[...], preferred_element_type=jnp.float32)
    upd = upd + bu_ref[...]
    norm = jnp.sqrt(jnp.sum(upd * upd, axis=1, keepdims=True) + 1e-8)
    proj = upd * jnp.minimum(1.0, (1.0 - 1e-5) / norm)

    def _cr(r):
        def dist(u, v):
            return jnp.sqrt(jnp.sum((u - v) ** 2) + 1e-8)
        a, b, cc, d = r[0:1, :], r[1:2, :], r[2:3, :], r[3:4, :]
        return (dist(a, cc) * dist(b, d)) / (dist(a, d) * dist(b, cc) + 1e-8)

    @pl.when(i == 0)
    def _():
        cr_init = _cr(x4_ref[0:4, :])
        cr_cur = _cr(proj[0:4, :])
        scale_smem[0] = cr_init / (cr_cur + 1e-8)

    out_ref[...] = proj * scale_smem[0]


def _tc_epilogue(px, pe, pc, x4, W_msg, W_edge, W_upd, bme, bu):
    return pl.pallas_call(
        _tc_body,
        out_shape=jax.ShapeDtypeStruct((N, D_OUT), jnp.float32),
        grid=(N // _B,),
        in_specs=[
            pl.BlockSpec((NC, _B, D_IN), lambda i: (0, i, 0)),
            pl.BlockSpec((NC, _B, D_EDGE), lambda i: (0, i, 0)),
            pl.BlockSpec((NC, _B, D_EDGE), lambda i: (0, i, 0)),
            pl.BlockSpec((8, D_IN), lambda i: (0, 0)),
            pl.BlockSpec((D_IN, D_OUT), lambda i: (0, 0)),
            pl.BlockSpec((D_EDGE, D_OUT), lambda i: (0, 0)),
            pl.BlockSpec((D_OUT, D_OUT), lambda i: (0, 0)),
            pl.BlockSpec((1, D_OUT), lambda i: (0, 0)),
            pl.BlockSpec((1, D_OUT), lambda i: (0, 0)),
        ],
        out_specs=pl.BlockSpec((_B, D_OUT), lambda i: (i, 0)),
        scratch_shapes=[pltpu.SMEM((1,), jnp.float32)],
    )(px, pe, pc, x4, W_msg, W_edge, W_upd, bme, bu)


def kernel(x, edge_index, edge_attr, W_msg, b_msg, W_edge, b_edge, W_upd, b_upd):
    row = edge_index[0]
    col = edge_index[1]
    zero = jnp.zeros((C, D_IN), jnp.float32)
    px, pe, pc = _sc_segment_sums(x, row, col, edge_attr, zero)
    bme = (b_msg + b_edge).reshape(1, D_OUT)
    bu = b_upd.reshape(1, D_OUT)
    return _tc_epilogue(px, pe, pc, x, W_msg, W_edge, W_upd, bme, bu)
